# baseline (speedup 1.0000x reference)
import functools

import jax
import jax.numpy as jnp
from jax import lax
from jax.experimental import pallas as pl
from jax.experimental.pallas import tpu as pltpu
from jax.experimental.pallas import tpu_sc as plsc

EMBED_DIM = 384
MAX_POS = 64
D3 = EMBED_DIM // 3

NUM_CORES = 2
NUM_SUBCORES = 16
NUM_WORKERS = NUM_CORES * NUM_SUBCORES

TOKENS = 4 * 16384
TOK_PER_W = TOKENS // NUM_WORKERS
CHUNK = 64
NCHUNK = TOK_PER_W // CHUNK
GROUPS = CHUNK // 16
TBL_ROWS = 3 * MAX_POS
JCHUNKS = EMBED_DIM // 16
JCHUNKS32 = EMBED_DIM // 32


def _project_tables_body(d_ref, h_ref, w_ref, pwt_ref, pb_ref, out_ref):
    bias = pb_ref[0, :]
    td = jnp.dot(d_ref[...], pwt_ref[0:D3, :], preferred_element_type=jnp.float32)
    th = jnp.dot(h_ref[...], pwt_ref[D3:2 * D3, :], preferred_element_type=jnp.float32)
    tw = jnp.dot(w_ref[...], pwt_ref[2 * D3:3 * D3, :], preferred_element_type=jnp.float32)
    out_ref[0:MAX_POS, :] = td + bias[None, :]
    out_ref[MAX_POS:2 * MAX_POS, :] = th
    out_ref[2 * MAX_POS:3 * MAX_POS, :] = tw


def _project_tables(d_table, h_table, w_table, proj_w, proj_b):
    return pl.pallas_call(
        _project_tables_body,
        out_shape=jax.ShapeDtypeStruct((TBL_ROWS, EMBED_DIM), jnp.float32),
    )(d_table, h_table, w_table, proj_w.T, proj_b.reshape(1, EMBED_DIM))


def _sc_body(pos_hbm, tc_hbm, out_hbm, posv, tcv, ob0, ob1, sem0, sem1):
    wid = lax.axis_index("s") * NUM_CORES + lax.axis_index("c")
    base = wid * TOK_PER_W

    pltpu.sync_copy(pos_hbm.at[pl.ds(base * 3, TOK_PER_W * 3)], posv.at[pl.ds(0, TOK_PER_W * 3)])
    pltpu.sync_copy(tc_hbm, tcv)

    iota16 = lax.iota(jnp.int32, 16)
    obufs = (ob0, ob1)
    sems = (sem0, sem1)

    def process_chunk(c, obuf):
        def extract(k):
            pv = posv[pl.ds((c * CHUNK + k) * 3, 16)]
            return (
                jnp.clip(pv[0], 0, MAX_POS - 1) * (EMBED_DIM // 2),
                (jnp.clip(pv[1], 0, MAX_POS - 1) + MAX_POS) * (EMBED_DIM // 2),
                (jnp.clip(pv[2], 0, MAX_POS - 1) + 2 * MAX_POS) * (EMBED_DIM // 2),
            )

        def tok_body(k, carry):
            rd, rh, rw = carry
            nxt = extract(k + 1)
            toff = k * EMBED_DIM

            def flush(item):
                d16, h16, w16, off = item
                s32 = (plsc.bitcast(d16, jnp.bfloat16)
                       + plsc.bitcast(h16, jnp.bfloat16)
                       + plsc.bitcast(w16, jnp.bfloat16))
                sa, sb = plsc.unpack(s32, format=plsc.PackFormat.INTERLEAVED)
                obuf[pl.ds(toff + off, 16)] = sa
                obuf[pl.ds(toff + off + 16, 16)] = sb

            pend = []
            for jc in range(JCHUNKS32):
                j16w = jc * 16
                pend.append((
                    tcv[pl.ds(rd + j16w, 16)],
                    tcv[pl.ds(rh + j16w, 16)],
                    tcv[pl.ds(rw + j16w, 16)],
                    jc * 32,
                ))
                if len(pend) > 2:
                    flush(pend.pop(0))
            for item in pend:
                flush(item)
            return nxt

        lax.fori_loop(0, CHUNK, tok_body, extract(0))

    def chunk_pair(cp, carry):
        for b in range(2):
            c = cp * 2 + b

            @pl.when(cp > 0)
            def _wait_prev():
                pltpu.make_async_copy(
                    obufs[b], out_hbm.at[pl.ds(base * EMBED_DIM, CHUNK * EMBED_DIM)],
                    sems[b],
                ).wait()

            process_chunk(c, obufs[b])

            pltpu.make_async_copy(
                obufs[b],
                out_hbm.at[pl.ds((base + c * CHUNK) * EMBED_DIM, CHUNK * EMBED_DIM)],
                sems[b],
            ).start()
        return carry

    lax.fori_loop(0, NCHUNK // 2, chunk_pair, 0)

    for b in range(2):
        pltpu.make_async_copy(
            obufs[b], out_hbm.at[pl.ds(base * EMBED_DIM, CHUNK * EMBED_DIM)], sems[b]
        ).wait()


_sc_gather = functools.partial(
    pl.kernel,
    out_type=jax.ShapeDtypeStruct((TOKENS * EMBED_DIM,), jnp.float32),
    mesh=plsc.VectorSubcoreMesh(
        core_axis_name="c", subcore_axis_name="s",
        num_cores=NUM_CORES, num_subcores=NUM_SUBCORES,
    ),
    compiler_params=pltpu.CompilerParams(needs_layout_passes=False),
    scratch_types=[
        pltpu.VMEM((TOK_PER_W * 3 + 16,), jnp.int32),
        pltpu.VMEM((TBL_ROWS * EMBED_DIM // 2,), jnp.int32),
        pltpu.VMEM((CHUNK * EMBED_DIM,), jnp.float32),
        pltpu.VMEM((CHUNK * EMBED_DIM,), jnp.float32),
        pltpu.SemaphoreType.DMA,
        pltpu.SemaphoreType.DMA,
    ],
)(_sc_body)


@jax.jit
def kernel(positions, d_table, h_table, w_table, proj_w, proj_b):
    tc = _project_tables(d_table, h_table, w_table, proj_w, proj_b)
    tcb = (
        tc.astype(jnp.bfloat16)
        .reshape(TBL_ROWS, JCHUNKS32, 2, 16)
        .transpose(0, 1, 3, 2)
        .reshape(TBL_ROWS * EMBED_DIM // 2, 2)
    )
    tci = lax.bitcast_convert_type(tcb, jnp.int32)
    pos = positions.astype(jnp.int32).reshape(TOKENS * 3)
    out = _sc_gather(pos, tci)
    return out.reshape(positions.shape[0], positions.shape[1], EMBED_DIM)


# speedup vs baseline: 1.6757x; 1.6757x over previous
import functools

import jax
import jax.numpy as jnp
from jax import lax
from jax.experimental import pallas as pl
from jax.experimental.pallas import tpu as pltpu
from jax.experimental.pallas import tpu_sc as plsc

EMBED_DIM = 384
MAX_POS = 64
D3 = EMBED_DIM // 3

NUM_CORES = 2
NUM_SUBCORES = 16
NUM_WORKERS = NUM_CORES * NUM_SUBCORES

TOKENS = 4 * 16384
TOK_PER_W = TOKENS // NUM_WORKERS
CHUNK = 64
NCHUNK = TOK_PER_W // CHUNK
GROUPS = CHUNK // 16
TBL_ROWS = 3 * MAX_POS
JCHUNKS = EMBED_DIM // 16
JCHUNKS32 = EMBED_DIM // 32


def _project_tables_body(d_ref, h_ref, w_ref, pwt_ref, pb_ref, out_ref):
    bias = pb_ref[0, :]
    td = jnp.dot(d_ref[...], pwt_ref[0:D3, :], preferred_element_type=jnp.float32)
    th = jnp.dot(h_ref[...], pwt_ref[D3:2 * D3, :], preferred_element_type=jnp.float32)
    tw = jnp.dot(w_ref[...], pwt_ref[2 * D3:3 * D3, :], preferred_element_type=jnp.float32)
    out_ref[0:MAX_POS, :] = td + bias[None, :]
    out_ref[MAX_POS:2 * MAX_POS, :] = th
    out_ref[2 * MAX_POS:3 * MAX_POS, :] = tw


def _project_tables(d_table, h_table, w_table, proj_w, proj_b):
    return pl.pallas_call(
        _project_tables_body,
        out_shape=jax.ShapeDtypeStruct((TBL_ROWS, EMBED_DIM), jnp.float32),
    )(d_table, h_table, w_table, proj_w.T, proj_b.reshape(1, EMBED_DIM))


def _sc_body(pos_hbm, tc_hbm, out_hbm, posv, tcv, ob0, ob1, sem0, sem1):
    wid = lax.axis_index("s") * NUM_CORES + lax.axis_index("c")
    base = wid * TOK_PER_W
    bb = wid // 8
    nb = (wid % 8) * TOK_PER_W

    pltpu.sync_copy(pos_hbm.at[pl.ds(base * 3, TOK_PER_W * 3)], posv.at[pl.ds(0, TOK_PER_W * 3)])
    pltpu.sync_copy(tc_hbm, tcv)

    iota16 = lax.iota(jnp.int32, 16)
    obufs = (ob0, ob1)
    sems = (sem0, sem1)

    def process_chunk(c, obuf):
        def extract(k):
            pv = posv[pl.ds((c * CHUNK + k) * 3, 16)]
            return (
                jnp.clip(pv[0], 0, MAX_POS - 1) * (EMBED_DIM // 2),
                (jnp.clip(pv[1], 0, MAX_POS - 1) + MAX_POS) * (EMBED_DIM // 2),
                (jnp.clip(pv[2], 0, MAX_POS - 1) + 2 * MAX_POS) * (EMBED_DIM // 2),
            )

        def tok_body(k, carry):
            rd, rh, rw = carry
            nxt = extract(k + 1)
            tok = k

            def flush(item):
                d16, h16, w16, off = item
                s32 = (plsc.bitcast(d16, jnp.bfloat16)
                       + plsc.bitcast(h16, jnp.bfloat16)
                       + plsc.bitcast(w16, jnp.bfloat16))
                sa, sb = plsc.unpack(s32, format=plsc.PackFormat.INTERLEAVED)
                obuf[tok, pl.ds(off, 16)] = sa
                obuf[tok, pl.ds(off + 16, 16)] = sb

            pend = []
            for jc in range(JCHUNKS32):
                j16w = jc * 16
                pend.append((
                    tcv[pl.ds(rd + j16w, 16)],
                    tcv[pl.ds(rh + j16w, 16)],
                    tcv[pl.ds(rw + j16w, 16)],
                    jc * 32,
                ))
                if len(pend) > 2:
                    flush(pend.pop(0))
            for item in pend:
                flush(item)
            return nxt

        lax.fori_loop(0, CHUNK, tok_body, extract(0))

    def chunk_pair(cp, carry):
        for b in range(2):
            c = cp * 2 + b

            @pl.when(cp > 0)
            def _wait_prev():
                pltpu.make_async_copy(
                    obufs[b], out_hbm.at[bb, pl.ds(nb, CHUNK)],
                    sems[b],
                ).wait()

            process_chunk(c, obufs[b])

            pltpu.make_async_copy(
                obufs[b],
                out_hbm.at[bb, pl.ds(nb + c * CHUNK, CHUNK)],
                sems[b],
            ).start()
        return carry

    lax.fori_loop(0, NCHUNK // 2, chunk_pair, 0)

    for b in range(2):
        pltpu.make_async_copy(
            obufs[b], out_hbm.at[bb, pl.ds(nb, CHUNK)], sems[b]
        ).wait()


_sc_gather = functools.partial(
    pl.kernel,
    out_type=jax.ShapeDtypeStruct((4, 16384, EMBED_DIM), jnp.float32),
    mesh=plsc.VectorSubcoreMesh(
        core_axis_name="c", subcore_axis_name="s",
        num_cores=NUM_CORES, num_subcores=NUM_SUBCORES,
    ),
    compiler_params=pltpu.CompilerParams(needs_layout_passes=False),
    scratch_types=[
        pltpu.VMEM((TOK_PER_W * 3 + 16,), jnp.int32),
        pltpu.VMEM((TBL_ROWS * EMBED_DIM // 2,), jnp.int32),
        pltpu.VMEM((CHUNK, EMBED_DIM), jnp.float32),
        pltpu.VMEM((CHUNK, EMBED_DIM), jnp.float32),
        pltpu.SemaphoreType.DMA,
        pltpu.SemaphoreType.DMA,
    ],
)(_sc_body)


@jax.jit
def kernel(positions, d_table, h_table, w_table, proj_w, proj_b):
    tc = _project_tables(d_table, h_table, w_table, proj_w, proj_b)
    tcb = (
        tc.astype(jnp.bfloat16)
        .reshape(TBL_ROWS, JCHUNKS32, 2, 16)
        .transpose(0, 1, 3, 2)
        .reshape(TBL_ROWS * EMBED_DIM // 2, 2)
    )
    tci = lax.bitcast_convert_type(tcb, jnp.int32)
    pos = positions.astype(jnp.int32).reshape(TOKENS * 3)
    return _sc_gather(pos, tci)


# speedup vs baseline: 1.7808x; 1.0628x over previous
import functools

import jax
import jax.numpy as jnp
from jax import lax
from jax.experimental import pallas as pl
from jax.experimental.pallas import tpu as pltpu
from jax.experimental.pallas import tpu_sc as plsc

EMBED_DIM = 384
MAX_POS = 64
D3 = EMBED_DIM // 3

NUM_CORES = 2
NUM_SUBCORES = 16
NUM_WORKERS = NUM_CORES * NUM_SUBCORES

TOKENS = 4 * 16384
TOK_PER_W = TOKENS // NUM_WORKERS
CHUNK = 64
NCHUNK = TOK_PER_W // CHUNK
GROUPS = CHUNK // 16
TBL_ROWS = 3 * MAX_POS
JCHUNKS = EMBED_DIM // 16
JCHUNKS32 = EMBED_DIM // 32


def _project_tables_body(d_ref, h_ref, w_ref, pwt_ref, pb_ref, out_ref):
    bias = pb_ref[0, :]
    td = jnp.dot(d_ref[...], pwt_ref[0:D3, :], preferred_element_type=jnp.float32)
    th = jnp.dot(h_ref[...], pwt_ref[D3:2 * D3, :], preferred_element_type=jnp.float32)
    tw = jnp.dot(w_ref[...], pwt_ref[2 * D3:3 * D3, :], preferred_element_type=jnp.float32)
    out_ref[0:MAX_POS, :] = td + bias[None, :]
    out_ref[MAX_POS:2 * MAX_POS, :] = th
    out_ref[2 * MAX_POS:3 * MAX_POS, :] = tw


def _project_tables(d_table, h_table, w_table, proj_w, proj_b):
    return pl.pallas_call(
        _project_tables_body,
        out_shape=jax.ShapeDtypeStruct((TBL_ROWS, EMBED_DIM), jnp.float32),
    )(d_table, h_table, w_table, proj_w.T, proj_b.reshape(1, EMBED_DIM))


def _sc_body(pos_hbm, tc_hbm, out_hbm, posv, tcv, ob0, ob1, sem0, sem1):
    wid = lax.axis_index("s") * NUM_CORES + lax.axis_index("c")
    base = wid * TOK_PER_W
    bb = wid // 8
    nb = (wid % 8) * TOK_PER_W

    pltpu.sync_copy(pos_hbm.at[pl.ds(base * 3, TOK_PER_W * 3)], posv.at[pl.ds(0, TOK_PER_W * 3)])
    pltpu.sync_copy(tc_hbm, tcv)

    iota16 = lax.iota(jnp.int32, 16)
    obufs = (ob0, ob1)
    sems = (sem0, sem1)

    def process_chunk(c, obuf):
        def extract(p):
            pv = posv[pl.ds((c * CHUNK + 2 * p) * 3, 16)]
            out = []
            for t in range(2):
                out.append((
                    jnp.clip(pv[3 * t], 0, MAX_POS - 1) * (EMBED_DIM // 2),
                    (jnp.clip(pv[3 * t + 1], 0, MAX_POS - 1) + MAX_POS) * (EMBED_DIM // 2),
                    (jnp.clip(pv[3 * t + 2], 0, MAX_POS - 1) + 2 * MAX_POS) * (EMBED_DIM // 2),
                ))
            return out

        def pair_body(p, carry):
            nxt = extract(p + 1)

            def flush(item):
                tok, d16, h16, w16, off = item
                s32 = (plsc.bitcast(d16, jnp.bfloat16)
                       + plsc.bitcast(h16, jnp.bfloat16)
                       + plsc.bitcast(w16, jnp.bfloat16))
                sa, sb = plsc.unpack(s32, format=plsc.PackFormat.INTERLEAVED)
                obuf[tok, pl.ds(off, 16)] = sa
                obuf[tok, pl.ds(off + 16, 16)] = sb

            pend = []
            for t in range(2):
                rd, rh, rw = carry[t]
                tok = 2 * p + t
                for jc in range(JCHUNKS32):
                    j16w = jc * 16
                    pend.append((
                        tok,
                        tcv[pl.ds(rd + j16w, 16)],
                        tcv[pl.ds(rh + j16w, 16)],
                        tcv[pl.ds(rw + j16w, 16)],
                        jc * 32,
                    ))
                    if len(pend) > 2:
                        flush(pend.pop(0))
            for item in pend:
                flush(item)
            return nxt

        lax.fori_loop(0, CHUNK // 2, pair_body, extract(0))

    def chunk_pair(cp, carry):
        for b in range(2):
            c = cp * 2 + b

            @pl.when(cp > 0)
            def _wait_prev():
                pltpu.make_async_copy(
                    obufs[b], out_hbm.at[bb, pl.ds(nb, CHUNK)],
                    sems[b],
                ).wait()

            process_chunk(c, obufs[b])

            pltpu.make_async_copy(
                obufs[b],
                out_hbm.at[bb, pl.ds(nb + c * CHUNK, CHUNK)],
                sems[b],
            ).start()
        return carry

    lax.fori_loop(0, NCHUNK // 2, chunk_pair, 0)

    for b in range(2):
        pltpu.make_async_copy(
            obufs[b], out_hbm.at[bb, pl.ds(nb, CHUNK)], sems[b]
        ).wait()


_sc_gather = functools.partial(
    pl.kernel,
    out_type=jax.ShapeDtypeStruct((4, 16384, EMBED_DIM), jnp.float32),
    mesh=plsc.VectorSubcoreMesh(
        core_axis_name="c", subcore_axis_name="s",
        num_cores=NUM_CORES, num_subcores=NUM_SUBCORES,
    ),
    compiler_params=pltpu.CompilerParams(needs_layout_passes=False),
    scratch_types=[
        pltpu.VMEM((TOK_PER_W * 3 + 16,), jnp.int32),
        pltpu.VMEM((TBL_ROWS * EMBED_DIM // 2,), jnp.int32),
        pltpu.VMEM((CHUNK, EMBED_DIM), jnp.float32),
        pltpu.VMEM((CHUNK, EMBED_DIM), jnp.float32),
        pltpu.SemaphoreType.DMA,
        pltpu.SemaphoreType.DMA,
    ],
)(_sc_body)


@jax.jit
def kernel(positions, d_table, h_table, w_table, proj_w, proj_b):
    tc = _project_tables(d_table, h_table, w_table, proj_w, proj_b)
    tcb = (
        tc.astype(jnp.bfloat16)
        .reshape(TBL_ROWS, JCHUNKS32, 2, 16)
        .transpose(0, 1, 3, 2)
        .reshape(TBL_ROWS * EMBED_DIM // 2, 2)
    )
    tci = lax.bitcast_convert_type(tcb, jnp.int32)
    pos = positions.astype(jnp.int32).reshape(TOKENS * 3)
    return _sc_gather(pos, tci)


# speedup vs baseline: 1.8311x; 1.0282x over previous
import functools

import jax
import jax.numpy as jnp
from jax import lax
from jax.experimental import pallas as pl
from jax.experimental.pallas import tpu as pltpu
from jax.experimental.pallas import tpu_sc as plsc

EMBED_DIM = 384
MAX_POS = 64
D3 = EMBED_DIM // 3

NUM_CORES = 2
NUM_SUBCORES = 16
NUM_WORKERS = NUM_CORES * NUM_SUBCORES

TOKENS = 4 * 16384
TOK_PER_W = TOKENS // NUM_WORKERS
CHUNK = 64
NCHUNK = TOK_PER_W // CHUNK
GROUPS = CHUNK // 16
TBL_ROWS = 3 * MAX_POS
JCHUNKS = EMBED_DIM // 16
JCHUNKS32 = EMBED_DIM // 32


def _project_tables_body(d_ref, h_ref, w_ref, pwt_ref, pb_ref, out_ref):
    bias = pb_ref[0, :]
    td = jnp.dot(d_ref[...], pwt_ref[0:D3, :], preferred_element_type=jnp.float32)
    th = jnp.dot(h_ref[...], pwt_ref[D3:2 * D3, :], preferred_element_type=jnp.float32)
    tw = jnp.dot(w_ref[...], pwt_ref[2 * D3:3 * D3, :], preferred_element_type=jnp.float32)
    out_ref[0:MAX_POS, :] = td + bias[None, :]
    out_ref[MAX_POS:2 * MAX_POS, :] = th
    out_ref[2 * MAX_POS:3 * MAX_POS, :] = tw


def _project_tables(d_table, h_table, w_table, proj_w, proj_b):
    return pl.pallas_call(
        _project_tables_body,
        out_shape=jax.ShapeDtypeStruct((TBL_ROWS, EMBED_DIM), jnp.float32),
    )(d_table, h_table, w_table, proj_w.T, proj_b.reshape(1, EMBED_DIM))


def _sc_body(pos_hbm, tc_hbm, out_hbm, posv, tcv, ob0, ob1, sem0, sem1):
    wid = lax.axis_index("s") * NUM_CORES + lax.axis_index("c")
    base = wid * TOK_PER_W
    bb = wid // 8
    nb = (wid % 8) * TOK_PER_W

    pltpu.sync_copy(pos_hbm.at[pl.ds(base * 3, TOK_PER_W * 3)], posv.at[pl.ds(0, TOK_PER_W * 3)])
    pltpu.sync_copy(tc_hbm, tcv)

    iota16 = lax.iota(jnp.int32, 16)
    obufs = (ob0, ob1)
    sems = (sem0, sem1)

    def process_chunk(c, obuf):
        def extract(p):
            pv = posv[pl.ds((c * CHUNK + 4 * p) * 3, 16)]
            out = []
            for t in range(4):
                out.append((
                    jnp.clip(pv[3 * t], 0, MAX_POS - 1) * (EMBED_DIM // 2),
                    (jnp.clip(pv[3 * t + 1], 0, MAX_POS - 1) + MAX_POS) * (EMBED_DIM // 2),
                    (jnp.clip(pv[3 * t + 2], 0, MAX_POS - 1) + 2 * MAX_POS) * (EMBED_DIM // 2),
                ))
            return out

        def pair_body(p, carry):
            nxt = extract(p + 1)

            def flush(item):
                tok, d16, h16, w16, off = item
                s32 = (plsc.bitcast(d16, jnp.bfloat16)
                       + plsc.bitcast(h16, jnp.bfloat16)
                       + plsc.bitcast(w16, jnp.bfloat16))
                sa, sb = plsc.unpack(s32, format=plsc.PackFormat.INTERLEAVED)
                obuf[tok, pl.ds(off, 16)] = sa
                obuf[tok, pl.ds(off + 16, 16)] = sb

            pend = []
            for t in range(4):
                rd, rh, rw = carry[t]
                tok = 4 * p + t
                for jc in range(JCHUNKS32):
                    j16w = jc * 16
                    pend.append((
                        tok,
                        tcv[pl.ds(rd + j16w, 16)],
                        tcv[pl.ds(rh + j16w, 16)],
                        tcv[pl.ds(rw + j16w, 16)],
                        jc * 32,
                    ))
                    if len(pend) > 2:
                        flush(pend.pop(0))
            for item in pend:
                flush(item)
            return nxt

        lax.fori_loop(0, CHUNK // 4, pair_body, extract(0))

    def chunk_pair(cp, carry):
        for b in range(2):
            c = cp * 2 + b

            @pl.when(cp > 0)
            def _wait_prev():
                pltpu.make_async_copy(
                    obufs[b], out_hbm.at[bb, pl.ds(nb, CHUNK)],
                    sems[b],
                ).wait()

            process_chunk(c, obufs[b])

            pltpu.make_async_copy(
                obufs[b],
                out_hbm.at[bb, pl.ds(nb + c * CHUNK, CHUNK)],
                sems[b],
            ).start()
        return carry

    lax.fori_loop(0, NCHUNK // 2, chunk_pair, 0)

    for b in range(2):
        pltpu.make_async_copy(
            obufs[b], out_hbm.at[bb, pl.ds(nb, CHUNK)], sems[b]
        ).wait()


_sc_gather = functools.partial(
    pl.kernel,
    out_type=jax.ShapeDtypeStruct((4, 16384, EMBED_DIM), jnp.float32),
    mesh=plsc.VectorSubcoreMesh(
        core_axis_name="c", subcore_axis_name="s",
        num_cores=NUM_CORES, num_subcores=NUM_SUBCORES,
    ),
    compiler_params=pltpu.CompilerParams(needs_layout_passes=False),
    scratch_types=[
        pltpu.VMEM((TOK_PER_W * 3 + 16,), jnp.int32),
        pltpu.VMEM((TBL_ROWS * EMBED_DIM // 2,), jnp.int32),
        pltpu.VMEM((CHUNK, EMBED_DIM), jnp.float32),
        pltpu.VMEM((CHUNK, EMBED_DIM), jnp.float32),
        pltpu.SemaphoreType.DMA,
        pltpu.SemaphoreType.DMA,
    ],
)(_sc_body)


@jax.jit
def kernel(positions, d_table, h_table, w_table, proj_w, proj_b):
    tc = _project_tables(d_table, h_table, w_table, proj_w, proj_b)
    tcb = (
        tc.astype(jnp.bfloat16)
        .reshape(TBL_ROWS, JCHUNKS32, 2, 16)
        .transpose(0, 1, 3, 2)
        .reshape(TBL_ROWS * EMBED_DIM // 2, 2)
    )
    tci = lax.bitcast_convert_type(tcb, jnp.int32)
    pos = positions.astype(jnp.int32).reshape(TOKENS * 3)
    return _sc_gather(pos, tci)
